# baseline (speedup 1.0000x reference)
import functools

import jax
import jax.numpy as jnp
from jax import lax
from jax.experimental import pallas as pl
from jax.experimental.pallas import tpu as pltpu
from jax.experimental.pallas import tpu_sc as plsc

P = 7
S = 2
SCALES = (0.25, 0.125, 0.0625, 0.03125)
CANON_SIZE = 224.0
CANON_LEVEL = 4
MIN_LEVEL, MAX_LEVEL = 2, 5
HWS = ((128, 128), (64, 64), (32, 32), (16, 16))
C = 256
CW = C // 2
NBOX_PAD = 1024
BPW = 32
NHALF = 112
NROW = 2 * NHALF
NBIN = P * P

_NC = 2
_NS = 16


@functools.lru_cache(maxsize=None)
def _build_gather_sc():
    mesh = plsc.VectorSubcoreMesh(core_axis_name="c", subcore_axis_name="s",
                                  num_cores=_NC, num_subcores=_NS)
    return functools.partial(
        pl.kernel,
        out_type=jax.ShapeDtypeStruct((NBOX_PAD, NROW, CW), jnp.int32),
        mesh=mesh,
        scratch_types=[
            pltpu.VMEM((NHALF,), jnp.int32),
            pltpu.VMEM((NHALF,), jnp.int32),
            pltpu.VMEM((NHALF,), jnp.int32),
            pltpu.VMEM((NHALF,), jnp.int32),
            pltpu.VMEM((2, NROW, CW), jnp.int32),
            pltpu.SemaphoreType.DMA,
            pltpu.SemaphoreType.DMA,
            pltpu.SemaphoreType.DMA,
        ],
    )(_gather_body)


def _gather_body(table_hbm, idx_hbm, rows_hbm, i00, i01, i10, i11,
                 rows_v, gsem, osem, isem):
    wid = lax.axis_index("s") * _NC + lax.axis_index("c")
    g0 = wid * BPW
    ih = ((i00, i01), (i10, i11))

    def idx_descs(bl, par):
        return [
            pltpu.make_async_copy(idx_hbm.at[2 * (g0 + bl) + h],
                                  ih[par][h], isem)
            for h in range(2)
        ]

    def gather_descs(par, buf):
        return [
            pltpu.make_async_copy(
                table_hbm.at[ih[par][h]],
                rows_v.at[buf, pl.ds(h * NHALF, NHALF)], gsem)
            for h in range(2)
        ]

    def out_desc(buf, gbox):
        return pltpu.make_async_copy(rows_v.at[buf], rows_hbm.at[gbox], osem)

    for cp in idx_descs(0, 0):
        cp.start()
    for cp in idx_descs(0, 0):
        cp.wait()
    for cp in gather_descs(0, 0):
        cp.start()
    for cp in idx_descs(1, 1):
        cp.start()

    def pair_body(i, carry):
        for par in range(2):
            bl = 2 * i + par

            @pl.when(bl + 1 < BPW)
            def _(par=par, bl=bl):
                @pl.when(bl >= 2)
                def _():
                    out_desc(1 - par, g0).wait()
                for cp in idx_descs(bl + 1, 1 - par):
                    cp.wait()
                for cp in gather_descs(1 - par, 1 - par):
                    cp.start()

            for cp in gather_descs(par, par):
                cp.wait()
            out_desc(par, g0 + bl).start()

            @pl.when(bl + 2 < BPW)
            def _(par=par, bl=bl):
                for cp in idx_descs(bl + 2, par):
                    cp.start()
        return carry

    lax.fori_loop(0, BPW // 2, pair_body, 0)
    out_desc(0, g0).wait()
    out_desc(1, g0).wait()


_TCBB = 8


def _pool_tc_body(rows_ref, out_ref):
    v = rows_ref[...].astype(jnp.float32)
    m = jnp.maximum(
        jnp.maximum(v[:, 0, 0:NBIN], v[:, 0, NBIN:2 * NBIN]),
        jnp.maximum(v[:, 1, 0:NBIN], v[:, 1, NBIN:2 * NBIN]))
    out_ref[...] = m


@functools.lru_cache(maxsize=None)
def _build_pool_tc():
    return pl.pallas_call(
        _pool_tc_body,
        grid=(NBOX_PAD // _TCBB,),
        in_specs=[pl.BlockSpec((_TCBB, 2, NHALF, C), lambda i: (i, 0, 0, 0))],
        out_specs=pl.BlockSpec((_TCBB, NBIN, C), lambda i: (i, 0, 0)),
        out_shape=jax.ShapeDtypeStruct((NBOX_PAD, NBIN, C), jnp.float32),
    )


def _prep_indices(fmt):
    areas = (fmt[:, 3] - fmt[:, 1]) * (fmt[:, 4] - fmt[:, 2])
    sizes = jnp.sqrt(areas)
    levels = jnp.clip(
        jnp.floor(CANON_LEVEL + jnp.log2(sizes / CANON_SIZE + 1e-8)),
        MIN_LEVEL, MAX_LEVEL).astype(jnp.int32) - MIN_LEVEL
    k = fmt.shape[0]
    bidx = fmt[:, 0].astype(jnp.int32)
    offs = (jnp.arange(S, dtype=jnp.float32) + 0.5) / S
    pids = jnp.arange(P, dtype=jnp.float32)
    grid14 = (pids[:, None] + offs[None, :]).reshape(-1)

    flat_all = []
    row_off = 0
    for l in range(4):
        h, w = HWS[l]
        scale = SCALES[l]
        x1 = jnp.round(fmt[:, 1] * scale)
        y1 = jnp.round(fmt[:, 2] * scale)
        x2 = jnp.round(fmt[:, 3] * scale)
        y2 = jnp.round(fmt[:, 4] * scale)
        bw = jnp.maximum(x2 - x1, 1.0) / P
        bh = jnp.maximum(y2 - y1, 1.0) / P
        sy = y1[:, None] + grid14[None, :] * bh[:, None]
        sx = x1[:, None] + grid14[None, :] * bw[:, None]
        iy = jnp.clip(jnp.floor(sy), 0, h - 1).astype(jnp.int32)
        ix = jnp.clip(jnp.floor(sx), 0, w - 1).astype(jnp.int32)
        flat = (row_off + bidx[:, None, None] * (h * w)
                + iy[:, :, None] * w + ix[:, None, :])
        flat_all.append(flat)
        row_off += 2 * h * w
    flat = jnp.stack(flat_all, 1)
    flat = jnp.take_along_axis(
        flat, levels[:, None, None, None], axis=1)[:, 0]
    flat = flat.reshape(k, P, 2, P, 2)
    flat = jnp.transpose(flat, (0, 2, 4, 1, 3)).reshape(k, 2, 2 * NBIN)
    half = jnp.pad(flat, ((0, NBOX_PAD - k), (0, 0), (0, NHALF - 2 * NBIN)))
    return half.reshape(2 * NBOX_PAD, NHALF)


def kernel(feat_p2, feat_p3, feat_p4, feat_p5, boxes_img0, boxes_img1):
    box_lists = [boxes_img0, boxes_img1]
    fmt = jnp.concatenate(
        [jnp.concatenate([jnp.full((b.shape[0], 1), float(i), b.dtype), b],
                         axis=1)
         for i, b in enumerate(box_lists)], axis=0)
    k = fmt.shape[0]
    idx = _prep_indices(fmt)
    table = jnp.concatenate(
        [jnp.transpose(f, (0, 2, 3, 1)).reshape(-1, C)
         for f in (feat_p2, feat_p3, feat_p4, feat_p5)],
        axis=0).astype(jnp.bfloat16)
    table = lax.bitcast_convert_type(table.reshape(-1, CW, 2), jnp.int32)
    rows = _build_gather_sc()(table, idx)
    rows = lax.bitcast_convert_type(rows, jnp.bfloat16)
    rows = rows.reshape(NBOX_PAD, 2, NHALF, C)
    out = _build_pool_tc()(rows)
    out = out[:k].reshape(k, P, P, C)
    return jnp.transpose(out, (0, 3, 1, 2))


# speedup vs baseline: 2.5768x; 2.5768x over previous
import functools

import jax
import jax.numpy as jnp
from jax import lax
from jax.experimental import pallas as pl
from jax.experimental.pallas import tpu as pltpu
from jax.experimental.pallas import tpu_sc as plsc

P = 7
S = 2
SCALES = (0.25, 0.125, 0.0625, 0.03125)
CANON_SIZE = 224.0
CANON_LEVEL = 4
MIN_LEVEL, MAX_LEVEL = 2, 5
HWS = ((128, 128), (64, 64), (32, 32), (16, 16))
C = 256
NBOX_PAD = 1024
BPW = 32
NHALF = 104
NROW = 2 * NHALF
OUTROW = C * P * P

_NC = 2
_NS = 16


@functools.lru_cache(maxsize=None)
def _build_roipool_sc():
    mesh = plsc.VectorSubcoreMesh(core_axis_name="c", subcore_axis_name="s",
                                  num_cores=_NC, num_subcores=_NS)
    return functools.partial(
        pl.kernel,
        out_type=jax.ShapeDtypeStruct((NBOX_PAD, OUTROW), jnp.float32),
        mesh=mesh,
        scratch_types=[
            pltpu.VMEM((NHALF,), jnp.int32),
            pltpu.VMEM((NHALF,), jnp.int32),
            pltpu.VMEM((NHALF,), jnp.int32),
            pltpu.VMEM((NHALF,), jnp.int32),
            pltpu.VMEM((2, NROW, C), jnp.float32),
            pltpu.VMEM((OUTROW,), jnp.float32),
            pltpu.SemaphoreType.DMA,
            pltpu.SemaphoreType.DMA,
            pltpu.SemaphoreType.DMA,
        ],
    )(_roipool_body)


def _row(y14, x14):
    h = y14 // 7
    return h * NHALF + (y14 - 7 * h) * 14 + x14


def _roipool_body(table_hbm, idx_hbm, out_hbm, i00, i01, i10, i11,
                  rows_v, out_v, gsem, osem, isem):
    wid = lax.axis_index("s") * _NC + lax.axis_index("c")
    g0 = wid * BPW
    ih = ((i00, i01), (i10, i11))

    def idx_descs(bl, par):
        return [
            pltpu.make_async_copy(idx_hbm.at[2 * (g0 + bl) + h],
                                  ih[par][h], isem)
            for h in range(2)
        ]

    def gather_descs(par, buf):
        return [
            pltpu.make_async_copy(
                table_hbm.at[ih[par][h]],
                rows_v.at[buf, pl.ds(h * NHALF, NHALF)], gsem)
            for h in range(2)
        ]

    def compute_box(buf, gbox):
        def cbody(cc, _):
            off = cc * 16
            base = rows_v.at[buf]
            for py in range(P):
                for px in range(P):
                    r00 = _row(2 * py, 2 * px)
                    r01 = _row(2 * py, 2 * px + 1)
                    r10 = _row(2 * py + 1, 2 * px)
                    r11 = _row(2 * py + 1, 2 * px + 1)
                    m = jnp.maximum(
                        jnp.maximum(base[r00, pl.ds(off, 16)],
                                    base[r01, pl.ds(off, 16)]),
                        jnp.maximum(base[r10, pl.ds(off, 16)],
                                    base[r11, pl.ds(off, 16)]))
                    out_v[pl.ds((py * P + px) * C + off, 16)] = m
            return _

        lax.fori_loop(0, C // 16, cbody, 0)
        pltpu.make_async_copy(out_v, out_hbm.at[gbox], osem).start()

    for cp in idx_descs(0, 0):
        cp.start()
    for cp in idx_descs(0, 0):
        cp.wait()
    for cp in gather_descs(0, 0):
        cp.start()
    for cp in idx_descs(1, 1):
        cp.start()

    def pair_body(i, carry):
        for par in range(2):
            bl = 2 * i + par

            @pl.when(bl + 1 < BPW)
            def _(par=par, bl=bl):
                for cp in idx_descs(bl + 1, 1 - par):
                    cp.wait()
                for cp in gather_descs(1 - par, 1 - par):
                    cp.start()

            for cp in gather_descs(par, par):
                cp.wait()

            @pl.when(bl + 2 < BPW)
            def _(par=par, bl=bl):
                for cp in idx_descs(bl + 2, par):
                    cp.start()

            @pl.when(bl >= 1)
            def _():
                pltpu.make_async_copy(out_v, out_hbm.at[g0], osem).wait()

            compute_box(par, g0 + bl)
        return carry

    lax.fori_loop(0, BPW // 2, pair_body, 0)
    pltpu.make_async_copy(out_v, out_hbm.at[g0], osem).wait()


def _prep_indices(fmt):
    areas = (fmt[:, 3] - fmt[:, 1]) * (fmt[:, 4] - fmt[:, 2])
    sizes = jnp.sqrt(areas)
    levels = jnp.clip(
        jnp.floor(CANON_LEVEL + jnp.log2(sizes / CANON_SIZE + 1e-8)),
        MIN_LEVEL, MAX_LEVEL).astype(jnp.int32) - MIN_LEVEL
    k = fmt.shape[0]
    bidx = fmt[:, 0].astype(jnp.int32)
    offs = (jnp.arange(S, dtype=jnp.float32) + 0.5) / S
    pids = jnp.arange(P, dtype=jnp.float32)
    grid14 = (pids[:, None] + offs[None, :]).reshape(-1)

    flat_all = []
    row_off = 0
    for l in range(4):
        h, w = HWS[l]
        scale = SCALES[l]
        x1 = jnp.round(fmt[:, 1] * scale)
        y1 = jnp.round(fmt[:, 2] * scale)
        x2 = jnp.round(fmt[:, 3] * scale)
        y2 = jnp.round(fmt[:, 4] * scale)
        bw = jnp.maximum(x2 - x1, 1.0) / P
        bh = jnp.maximum(y2 - y1, 1.0) / P
        sy = y1[:, None] + grid14[None, :] * bh[:, None]
        sx = x1[:, None] + grid14[None, :] * bw[:, None]
        iy = jnp.clip(jnp.floor(sy), 0, h - 1).astype(jnp.int32)
        ix = jnp.clip(jnp.floor(sx), 0, w - 1).astype(jnp.int32)
        flat = (row_off + bidx[:, None, None] * (h * w)
                + iy[:, :, None] * w + ix[:, None, :])
        flat_all.append(flat)
        row_off += 2 * h * w
    flat = jnp.stack(flat_all, 1)
    flat = jnp.take_along_axis(
        flat, levels[:, None, None, None], axis=1)[:, 0]
    half = flat.reshape(k, 2, 7 * 14)
    half = jnp.pad(half, ((0, NBOX_PAD - k), (0, 0), (0, NHALF - 98)))
    return half.reshape(2 * NBOX_PAD, NHALF)


def kernel(feat_p2, feat_p3, feat_p4, feat_p5, boxes_img0, boxes_img1):
    box_lists = [boxes_img0, boxes_img1]
    fmt = jnp.concatenate(
        [jnp.concatenate([jnp.full((b.shape[0], 1), float(i), b.dtype), b],
                         axis=1)
         for i, b in enumerate(box_lists)], axis=0)
    k = fmt.shape[0]
    idx = _prep_indices(fmt)
    table = jnp.concatenate(
        [jnp.transpose(f, (0, 2, 3, 1)).reshape(-1, C)
         for f in (feat_p2, feat_p3, feat_p4, feat_p5)], axis=0)
    out = _build_roipool_sc()(table, idx)
    return jnp.transpose(out[:k].reshape(k, P, P, C), (0, 3, 1, 2))
